# baseline (speedup 1.0000x reference)
import functools

import jax
import jax.numpy as jnp
from jax import lax
from jax.experimental import pallas as pl
from jax.experimental.pallas import tpu as pltpu
from jax.experimental.pallas import tpu_sc as plsc

B, S, H, V = 4, 2048, 1024, 100000
NC, NS, L = 2, 16, 16
NW = NC * NS
SBLK = S // NW
PH = 2
PBLK = SBLK // PH
C = 16
NCC = PBLK // C
NCH = B * NCC
JW = H // L

_mesh = plsc.VectorSubcoreMesh(core_axis_name="c", subcore_axis_name="s")


@functools.partial(
    pl.kernel,
    mesh=_mesh,
    compiler_params=pltpu.CompilerParams(needs_layout_passes=False),
    out_type=jax.ShapeDtypeStruct((B * S, H), jnp.float32),
    scratch_types=[
        pltpu.VMEM((PBLK, H), jnp.float32),
        pltpu.VMEM((2, H), jnp.float32),
        pltpu.VMEM((B * PBLK,), jnp.int32),
        pltpu.VMEM((B * PBLK,), jnp.int32),
        pltpu.VMEM((C, H), jnp.float32),
        pltpu.VMEM((C, H), jnp.float32),
        pltpu.VMEM((C, H), jnp.float32),
        pltpu.SemaphoreType.DMA,
        pltpu.SemaphoreType.DMA,
        pltpu.SemaphoreType.DMA,
        pltpu.SemaphoreType.DMA,
        pltpu.SemaphoreType.DMA,
        pltpu.SemaphoreType.DMA,
        pltpu.SemaphoreType.DMA,
    ],
)
def _emb_kernel(ids_hbm, tt_hbm, w_hbm, p_hbm, t_hbm, out_hbm,
                pbuf, tvm, idv, ttv, wbuf0, wbuf1, wbuf2,
                sem_w0, sem_w1, sem_w2, sem_o0, sem_o1, sem_o2, sem_i):
    wid = lax.axis_index("s") * NC + lax.axis_index("c")
    wbufs = (wbuf0, wbuf1, wbuf2)
    sems_w = (sem_w0, sem_w1, sem_w2)
    sems_o = (sem_o0, sem_o1, sem_o2)

    pltpu.sync_copy(t_hbm, tvm)
    base_iota = lax.iota(jnp.int32, L)

    gathers = [None, None, None]
    stores = [None, None, None]

    for h in range(PH):
        s0 = (wid + NW * h) * PBLK
        id_copies = []
        for b in range(B):
            id_copies.append(pltpu.async_copy(
                ids_hbm.at[pl.ds(b * S + s0, PBLK)],
                idv.at[pl.ds(b * PBLK, PBLK)], sem_i))
            id_copies.append(pltpu.async_copy(
                tt_hbm.at[pl.ds(b * S + s0, PBLK)],
                ttv.at[pl.ds(b * PBLK, PBLK)], sem_i))
        for cp in id_copies:
            cp.wait()

        def launch(i):
            slot = i % 3
            if stores[slot] is not None:
                stores[slot].wait()
                stores[slot] = None
            gathers[slot] = pltpu.async_copy(
                w_hbm.at[idv.at[pl.ds(i * C, C)]], wbufs[slot],
                sems_w[slot])

        launch(0)
        launch(1)

        pltpu.sync_copy(p_hbm.at[pl.ds(s0, PBLK)], pbuf)

        for i in range(NCH):
            if i + 2 < NCH:
                launch(i + 2)
            slot = i % 3
            b, c = divmod(i, NCC)
            gathers[slot].wait()
            wb = wbufs[slot]
            tt_vec = ttv[pl.ds(i * C, C)]

            def row_body(r, rvec, wb=wb, c=c, tt_vec=tt_vec):
                ttr = jnp.max(jnp.where(base_iota == rvec, tt_vec, 0))

                def col_body(j, _):
                    for k in range(4):
                        col = pl.ds(j * (4 * L) + k * L, L)
                        wb[r, col] = (wb[r, col] + pbuf[c * C + r, col]
                                      + tvm[ttr, col])
                    return 0

                lax.fori_loop(0, JW // 4, col_body, 0, unroll=True)
                return rvec + 1

            lax.fori_loop(0, C, row_body,
                          jnp.zeros((L,), jnp.int32), unroll=False)
            off = b * S + s0 + c * C
            stores[slot] = pltpu.async_copy(wb, out_hbm.at[pl.ds(off, C)],
                                            sems_o[slot])
    for slot in range(3):
        if stores[slot] is not None:
            stores[slot].wait()


def kernel(input_ids, token_type_ids, word_embeddings, position_embeddings,
           token_type_embeddings):
    ids = input_ids.reshape(-1).astype(jnp.int32)
    tt = token_type_ids.reshape(-1).astype(jnp.int32)
    out = _emb_kernel(ids, tt, word_embeddings, position_embeddings,
                      token_type_embeddings)
    return out.reshape(B, S, H)


# speedup vs baseline: 1.0042x; 1.0042x over previous
import functools

import jax
import jax.numpy as jnp
from jax import lax
from jax.experimental import pallas as pl
from jax.experimental.pallas import tpu as pltpu
from jax.experimental.pallas import tpu_sc as plsc

B, S, H, V = 4, 2048, 1024, 100000
NC, NS, L = 2, 16, 16
NW = NC * NS
SBLK = S // NW
PH = 2
PBLK = SBLK // PH
C = 16
NCC = PBLK // C
NCH = B * NCC
JW = H // L

_mesh = plsc.VectorSubcoreMesh(core_axis_name="c", subcore_axis_name="s")


@functools.partial(
    pl.kernel,
    mesh=_mesh,
    compiler_params=pltpu.CompilerParams(needs_layout_passes=False),
    out_type=jax.ShapeDtypeStruct((B * S, H), jnp.float32),
    scratch_types=[
        pltpu.VMEM((PBLK, H), jnp.float32),
        pltpu.VMEM((2, H), jnp.float32),
        pltpu.VMEM((B * PBLK,), jnp.int32),
        pltpu.VMEM((B * PBLK,), jnp.int32),
        pltpu.VMEM((C, H), jnp.float32),
        pltpu.VMEM((C, H), jnp.float32),
        pltpu.SemaphoreType.DMA,
        pltpu.SemaphoreType.DMA,
        pltpu.SemaphoreType.DMA,
        pltpu.SemaphoreType.DMA,
        pltpu.SemaphoreType.DMA,
    ],
)
def _emb_kernel(ids_hbm, tt_hbm, w_hbm, p_hbm, t_hbm, out_hbm,
                pbuf, tvm, idv, ttv, wbuf0, wbuf1,
                sem_w0, sem_w1, sem_o0, sem_o1, sem_i):
    wid = lax.axis_index("s") * NC + lax.axis_index("c")
    wbufs = (wbuf0, wbuf1)
    sems_w = (sem_w0, sem_w1)
    sems_o = (sem_o0, sem_o1)

    pltpu.sync_copy(t_hbm, tvm)
    base_iota = lax.iota(jnp.int32, L)

    gathers = [None, None]
    stores = [None, None]

    for h in range(PH):
        s0 = (wid + NW * h) * PBLK
        id_copies = []
        for b in range(B):
            id_copies.append(pltpu.async_copy(
                ids_hbm.at[pl.ds(b * S + s0, PBLK)],
                idv.at[pl.ds(b * PBLK, PBLK)], sem_i))
            id_copies.append(pltpu.async_copy(
                tt_hbm.at[pl.ds(b * S + s0, PBLK)],
                ttv.at[pl.ds(b * PBLK, PBLK)], sem_i))
        for cp in id_copies:
            cp.wait()

        def launch(i):
            slot = i % 2
            if stores[slot] is not None:
                stores[slot].wait()
                stores[slot] = None
            gathers[slot] = pltpu.async_copy(
                w_hbm.at[idv.at[pl.ds(i * C, C)]], wbufs[slot],
                sems_w[slot])

        launch(0)

        pltpu.sync_copy(p_hbm.at[pl.ds(s0, PBLK)], pbuf)

        for i in range(NCH):
            if i + 1 < NCH:
                launch(i + 1)
            slot = i % 2
            b, c = divmod(i, NCC)
            gathers[slot].wait()
            wb = wbufs[slot]
            tt_vec = ttv[pl.ds(i * C, C)]

            def row_body(r, rvec, wb=wb, c=c, tt_vec=tt_vec):
                ttr = jnp.max(jnp.where(base_iota == rvec, tt_vec, 0))

                def col_body(j, _):
                    for k in range(4):
                        col = pl.ds(j * (4 * L) + k * L, L)
                        wb[r, col] = (wb[r, col] + pbuf[c * C + r, col]
                                      + tvm[ttr, col])
                    return 0

                lax.fori_loop(0, JW // 4, col_body, 0, unroll=True)
                return rvec + 1

            lax.fori_loop(0, C, row_body,
                          jnp.zeros((L,), jnp.int32), unroll=False)
            off = b * S + s0 + c * C
            stores[slot] = pltpu.async_copy(wb, out_hbm.at[pl.ds(off, C)],
                                            sems_o[slot])
    for slot in range(2):
        if stores[slot] is not None:
            stores[slot].wait()


def kernel(input_ids, token_type_ids, word_embeddings, position_embeddings,
           token_type_embeddings):
    ids = input_ids.reshape(-1).astype(jnp.int32)
    tt = token_type_ids.reshape(-1).astype(jnp.int32)
    out = _emb_kernel(ids, tt, word_embeddings, position_embeddings,
                      token_type_embeddings)
    return out.reshape(B, S, H)
